# baseline (speedup 1.0000x reference)
import functools

import jax
import jax.numpy as jnp
from jax import lax
from jax.experimental import pallas as pl
from jax.experimental.pallas import tpu as pltpu
from jax.experimental.pallas import tpu_sc as plsc

B, N, C = 32, 8192, 128
K = 16
L = 16
NC = 2
TB = 32
SB = B - TB
WPB = 4
WROWS = N // WPB
HN = N // 2
RB = 256
NCH = WROWS // RB
GRP = RB // L
CW = HN // L

NEG = float("-inf")
IBIG = 2**31 - 1


def _scores_body(x_ref, o_ref):
    o_ref[...] = jnp.max(x_ref[...], axis=2)


def _tc_scores_half(x):
    return pl.pallas_call(
        _scores_body,
        grid=(TB // 8, 8),
        in_specs=[pl.BlockSpec((8, 1024, 128), lambda i, j: (i, j, 0))],
        out_specs=pl.BlockSpec((8, 1024), lambda i, j: (i, j)),
        out_shape=jax.ShapeDtypeStruct((TB, N), jnp.float32),
    )(x)


def _select_top16(scores_v, cand_v, cand_i, m, base, topv_v, topi_v, cw):
    lanes = lax.iota(jnp.int32, L)
    t0 = jnp.min(m)

    def p2(j, off):
        v = scores_v[pl.ds(j * L, L)]
        msk = v >= t0
        pos = off + plsc.cumsum(msk.astype(jnp.int32)) - 1
        plsc.store_scatter(cand_v, [pos], v, mask=msk)
        plsc.store_scatter(cand_i, [pos], base + j * L + lanes, mask=msk)
        return off + jnp.max(plsc.all_reduce_population_count(msk))

    c = lax.fori_loop(0, cw, p2, jnp.int32(0))

    pad_pos = jnp.full((L,), c, jnp.int32) + lanes
    plsc.store_scatter(cand_v, [pad_pos], jnp.full((L,), NEG, jnp.float32))
    plsc.store_scatter(cand_i, [pad_pos], jnp.full((L,), IBIG, jnp.int32))
    nch = (c + (L - 1)) // L
    lane0 = lanes == 0

    for s in range(K):
        def scan(j, carry):
            bv, bi, bp = carry
            v = cand_v[pl.ds(j * L, L)]
            ii = cand_i[pl.ds(j * L, L)]
            pp = lanes + j * L
            take = (v > bv) | ((v == bv) & (ii < bi))
            return (jnp.where(take, v, bv),
                    jnp.where(take, ii, bi),
                    jnp.where(take, pp, bp))

        bv, bi, bp = lax.fori_loop(
            0, nch, scan,
            (jnp.full((L,), NEG, jnp.float32),
             jnp.full((L,), IBIG, jnp.int32),
             jnp.full((L,), IBIG, jnp.int32)))
        mval = jnp.max(bv)
        eq = bv == mval
        mi = jnp.min(jnp.where(eq, bi, IBIG))
        pos = jnp.min(jnp.where(eq & (bi == mi), bp, IBIG))
        plsc.store_scatter(topv_v, [jnp.full((L,), s, jnp.int32)],
                           jnp.full((L,), mval, jnp.float32), mask=lane0)
        plsc.store_scatter(topi_v, [jnp.full((L,), s, jnp.int32)],
                           jnp.full((L,), mi, jnp.int32), mask=lane0)
        plsc.store_scatter(cand_v, [jnp.full((L,), pos, jnp.int32)],
                           jnp.full((L,), NEG, jnp.float32), mask=lane0)


def _merge_gather_mean(x2_hbm, out_row, mc_v, mc_i, idx_v, rows_v, out_v,
                       sem, nmc):
    lanes = lax.iota(jnp.int32, L)
    lane0 = lanes == 0
    for s in range(K):
        bv = jnp.full((L,), NEG, jnp.float32)
        bi = jnp.full((L,), IBIG, jnp.int32)
        bp = jnp.full((L,), IBIG, jnp.int32)
        for j in range(nmc):
            v = mc_v[pl.ds(j * L, L)]
            ii = mc_i[pl.ds(j * L, L)]
            pp = lanes + j * L
            take = (v > bv) | ((v == bv) & (ii < bi))
            bv = jnp.where(take, v, bv)
            bi = jnp.where(take, ii, bi)
            bp = jnp.where(take, pp, bp)
        mval = jnp.max(bv)
        eq = bv == mval
        mi = jnp.min(jnp.where(eq, bi, IBIG))
        pos = jnp.min(jnp.where(eq & (bi == mi), bp, IBIG))
        plsc.store_scatter(idx_v, [jnp.full((L,), s, jnp.int32)],
                           jnp.full((L,), mi, jnp.int32), mask=lane0)
        plsc.store_scatter(mc_v, [jnp.full((L,), pos, jnp.int32)],
                           jnp.full((L,), NEG, jnp.float32), mask=lane0)
    pltpu.async_copy(x2_hbm.at[idx_v], rows_v, sem).wait()
    for cc in range(C // L):
        acc = jnp.zeros((L,), jnp.float32)
        for r in range(K):
            acc = acc + rows_v[r, pl.ds(cc * L, L)]
        out_v[pl.ds(cc * L, L)] = acc * jnp.float32(1.0 / K)
    pltpu.sync_copy(out_v, out_row)


def _scb1_body(s1_hbm, cval_hbm, cidx_hbm,
               scores_v, cand_v, cand_i, topv_v, topi_v):
    cid = lax.axis_index("c")
    sid = lax.axis_index("s")
    w = sid * NC + cid
    for jj in range(TB * WPB // 32):
        job = w + jj * 32
        b = job // WPB
        q = job % WPB
        base_row = b * N + q * WROWS
        pltpu.sync_copy(s1_hbm.at[pl.ds(base_row, WROWS)], scores_v)

        def p1(j, m):
            return jnp.maximum(m, scores_v[pl.ds(j * L, L)])

        m = lax.fori_loop(0, WROWS // L, p1, jnp.full((L,), NEG, jnp.float32))
        _select_top16(scores_v, cand_v, cand_i, m, base_row,
                      topv_v, topi_v, WROWS // L)
        pltpu.sync_copy(topv_v, cval_hbm.at[pl.ds(job * K, K)])
        pltpu.sync_copy(topi_v, cidx_hbm.at[pl.ds(job * K, K)])


def _scm_body(cvB_hbm, ciB_hbm, x2_hbm, out_hbm,
              mc_v, mc_i, idx_v, rows_v, out_v, gsem):
    cid = lax.axis_index("c")
    sid = lax.axis_index("s")
    w = sid * NC + cid
    off = w * (WPB * K)
    pltpu.sync_copy(cvB_hbm.at[pl.ds(off, WPB * K)], mc_v)
    pltpu.sync_copy(ciB_hbm.at[pl.ds(off, WPB * K)], mc_i)
    _merge_gather_mean(x2_hbm, out_hbm.at[w], mc_v, mc_i,
                       idx_v, rows_v, out_v, gsem, WPB)


def _sel_scratch(nrows):
    return [
        pltpu.VMEM((nrows,), jnp.float32),
        pltpu.VMEM((nrows + L,), jnp.float32),
        pltpu.VMEM((nrows + L,), jnp.int32),
        pltpu.VMEM((K,), jnp.float32),
        pltpu.VMEM((K,), jnp.int32),
    ]


def _merge_scratch(nmc):
    return [
        pltpu.VMEM((nmc * K,), jnp.float32),
        pltpu.VMEM((nmc * K,), jnp.int32),
        pltpu.VMEM((K,), jnp.int32),
        pltpu.VMEM((K, C), jnp.float32),
        pltpu.VMEM((C,), jnp.float32),
        pltpu.SemaphoreType.DMA,
    ]

_MESH = dict(
    mesh=plsc.VectorSubcoreMesh(core_axis_name="c", subcore_axis_name="s"),
    compiler_params=pltpu.CompilerParams(needs_layout_passes=False),
)

_sc_b1 = functools.partial(
    pl.kernel,
    out_type=[jax.ShapeDtypeStruct((TB * WPB * K,), jnp.float32),
              jax.ShapeDtypeStruct((TB * WPB * K,), jnp.int32)],
    scratch_types=_sel_scratch(WROWS),
    **_MESH,
)(_scb1_body)

_sc_merge = functools.partial(
    pl.kernel,
    out_type=jax.ShapeDtypeStruct((B, C), jnp.float32),
    scratch_types=_merge_scratch(WPB),
    **_MESH,
)(_scm_body)


@jax.jit
def kernel(input):
    x2 = input.reshape(B * N, C)
    scores = _tc_scores_half(input)
    cvB, ciB = _sc_b1(scores.reshape(TB * N))
    out = _sc_merge(cvB, ciB, x2)
    return out[:, None, :]


# speedup vs baseline: 1.2244x; 1.2244x over previous
import functools

import jax
import jax.numpy as jnp
from jax import lax
from jax.experimental import pallas as pl
from jax.experimental.pallas import tpu as pltpu
from jax.experimental.pallas import tpu_sc as plsc

B, N, C = 32, 8192, 128
K = 16
L = 16
NC = 2
NCHUNKS = N // L

NEG = float("-inf")
IBIG = 2**31 - 1


def _scores_body(x_ref, o_ref):
    o_ref[...] = jnp.max(x_ref[...], axis=2)


def _tc_scores(x):
    return pl.pallas_call(
        _scores_body,
        grid=(4, 8),
        in_specs=[pl.BlockSpec((8, 1024, 128), lambda i, j: (i, j, 0))],
        out_specs=pl.BlockSpec((8, 1024), lambda i, j: (i, j)),
        out_shape=jax.ShapeDtypeStruct((B, N), jnp.float32),
    )(x)


def _sc_body(scores_hbm, x_hbm, out_hbm,
             scores_v, cand_v, cand_i, idx_v, rows_v, out_v, sem):
    cid = lax.axis_index("c")
    sid = lax.axis_index("s")
    b = sid * NC + cid
    lanes = lax.iota(jnp.int32, L)

    pltpu.sync_copy(scores_hbm.at[b], scores_v)

    def p1(j, m):
        return jnp.maximum(m, scores_v[pl.ds(j * L, L)])

    m = lax.fori_loop(0, NCHUNKS, p1, jnp.full((L,), NEG, jnp.float32))
    t0 = jnp.min(m)

    def p2(j, off):
        v = scores_v[pl.ds(j * L, L)]
        msk = v >= t0
        pos = off + plsc.cumsum(msk.astype(jnp.int32)) - 1
        idx = lanes + j * L
        plsc.store_scatter(cand_v, [pos], v, mask=msk)
        plsc.store_scatter(cand_i, [pos], idx, mask=msk)
        cnt = jnp.max(plsc.all_reduce_population_count(msk))
        return off + cnt

    c = lax.fori_loop(0, NCHUNKS, p2, jnp.int32(0))

    pad_pos = jnp.full((L,), c, jnp.int32) + lanes
    plsc.store_scatter(cand_v, [pad_pos], jnp.full((L,), NEG, jnp.float32))
    plsc.store_scatter(cand_i, [pad_pos], jnp.full((L,), IBIG, jnp.int32))
    nch = (c + (L - 1)) // L

    lane0 = lanes == 0
    for s in range(K):
        def scan(j, carry):
            bv, bi, bp = carry
            v = cand_v[pl.ds(j * L, L)]
            ii = cand_i[pl.ds(j * L, L)]
            pp = lanes + j * L
            take = v > bv
            return (jnp.where(take, v, bv),
                    jnp.where(take, ii, bi),
                    jnp.where(take, pp, bp))

        bv, bi, bp = lax.fori_loop(
            0, nch, scan,
            (jnp.full((L,), NEG, jnp.float32),
             jnp.full((L,), IBIG, jnp.int32),
             jnp.full((L,), IBIG, jnp.int32)))
        mval = jnp.max(bv)
        eq = bv == mval
        mi = jnp.min(jnp.where(eq, bi, IBIG))
        pos = jnp.min(jnp.where(eq & (bi == mi), bp, IBIG))
        plsc.store_scatter(idx_v, [jnp.full((L,), s, jnp.int32)],
                           jnp.full((L,), mi + b * N, jnp.int32), mask=lane0)
        plsc.store_scatter(cand_v, [jnp.full((L,), pos, jnp.int32)],
                           jnp.full((L,), NEG, jnp.float32), mask=lane0)

    pltpu.async_copy(x_hbm.at[idx_v], rows_v, sem).wait()
    for cc in range(C // L):
        acc = jnp.zeros((L,), jnp.float32)
        for r in range(K):
            acc = acc + rows_v[r, pl.ds(cc * L, L)]
        out_v[pl.ds(cc * L, L)] = acc * jnp.float32(1.0 / K)
    pltpu.sync_copy(out_v, out_hbm.at[b])


_sc_topk_mean = functools.partial(
    pl.kernel,
    mesh=plsc.VectorSubcoreMesh(core_axis_name="c", subcore_axis_name="s"),
    compiler_params=pltpu.CompilerParams(needs_layout_passes=False),
    out_type=jax.ShapeDtypeStruct((B, C), jnp.float32),
    scratch_types=[
        pltpu.VMEM((N,), jnp.float32),
        pltpu.VMEM((N + L,), jnp.float32),
        pltpu.VMEM((N + L,), jnp.int32),
        pltpu.VMEM((K,), jnp.int32),
        pltpu.VMEM((K, C), jnp.float32),
        pltpu.VMEM((C,), jnp.float32),
        pltpu.SemaphoreType.DMA,
    ],
)(_sc_body)


@jax.jit
def kernel(input):
    scores = _tc_scores(input)
    out = _sc_topk_mean(scores, input.reshape(B * N, C))
    return out[:, None, :]
